# baseline (speedup 1.0000x reference)
import functools

import jax
import jax.numpy as jnp
from jax import lax
from jax.experimental import pallas as pl
from jax.experimental.pallas import tpu as pltpu
from jax.experimental.pallas import tpu_sc as plsc

B = 4096
HIST = 200
HIDDEN = 16
USER_VOCAB = 1000
AD_VOCAB = 1000000
N_USER = 8
N_AD = 5
SEQ_TABLES = (1, 4)

NC = 2
NS = 16
NW = NC * NS

G = 128

U_ROWS = B * N_USER // G
A_ROWS = B * N_AD // G
S_ROWS = B * len(SEQ_TABLES) * HIST // G
U_PW = U_ROWS // NW
A_PW = 8
A_WORKERS = A_ROWS // A_PW
S_PW = S_ROWS // NW
SK = 10
NCHUNK = S_PW // SK
NPAIR = NCHUNK // 2


def _gather_body(user_tab, ad_tab, uidx, aidx, sidx,
                 u_out, a_out, s_out,
                 idx_all, rows0, rows1, sm_idx, sm_rows,
                 g0, g1, w0, w1, gsm, wsm):
    wid = lax.axis_index("c") * NS + lax.axis_index("s")
    sbase = wid * S_PW

    ub = wid * U_PW
    pltpu.sync_copy(uidx.at[pl.ds(ub, U_PW)], sm_idx)
    for j in range(U_PW):
        pltpu.async_copy(user_tab.at[sm_idx.at[j]], sm_rows.at[j], gsm)
    pltpu.make_async_copy(u_out.at[pl.ds(0, U_PW)], sm_rows, gsm).wait()
    pltpu.async_copy(sm_rows, u_out.at[pl.ds(ub, U_PW)], wsm)

    @pl.when(wid < A_WORKERS)
    def _():
        pltpu.make_async_copy(sm_rows, u_out.at[pl.ds(0, U_PW)], wsm).wait()
        ab = wid * A_PW
        pltpu.sync_copy(aidx.at[pl.ds(ab, A_PW)], sm_idx)
        for j in range(A_PW):
            pltpu.async_copy(ad_tab.at[sm_idx.at[j]], sm_rows.at[j], gsm)
        pltpu.make_async_copy(a_out.at[pl.ds(0, A_PW)], sm_rows, gsm).wait()
        pltpu.async_copy(sm_rows, a_out.at[pl.ds(ab, A_PW)], wsm)

    pltpu.sync_copy(sidx.at[pl.ds(sbase, S_PW)], idx_all)

    def fire(c, rows, sem):
        for j in range(SK):
            pltpu.async_copy(ad_tab.at[idx_all.at[c * SK + j]],
                             rows.at[j], sem)

    def drain_g(rows, sem):
        pltpu.make_async_copy(s_out.at[pl.ds(0, SK)], rows, sem).wait()

    fire(0, rows0, g0)
    fire(1, rows1, g1)

    def body(i, carry):
        for c, rows, gs, ws in ((2 * i, rows0, g0, w0),
                                (2 * i + 1, rows1, g1, w1)):
            out_slc = s_out.at[pl.ds(sbase + c * SK, SK)]
            drain_g(rows, gs)
            pltpu.async_copy(rows, out_slc, ws)
            pltpu.make_async_copy(rows, out_slc, ws).wait()

            @pl.when(c + 2 < NCHUNK)
            def _():
                fire(c + 2, rows, gs)
        return carry

    lax.fori_loop(0, NPAIR, body, 0)

    pltpu.make_async_copy(sm_rows, u_out.at[pl.ds(0, U_PW)], wsm).wait()


def kernel(user_indices, ad_indices, buy_seq_indices, user_tables, ad_tables):
    user_flat = user_tables.reshape(N_USER * USER_VOCAB, HIDDEN)
    ad_flat = ad_tables.reshape(N_AD * AD_VOCAB, HIDDEN)

    uidx = (user_indices
            + jnp.arange(N_USER, dtype=jnp.int32) * USER_VOCAB
            ).reshape(U_ROWS, G)
    aidx = (ad_indices
            + jnp.arange(N_AD, dtype=jnp.int32) * AD_VOCAB
            ).reshape(A_ROWS, G)
    soff = jnp.array(SEQ_TABLES, dtype=jnp.int32) * AD_VOCAB
    sidx = (buy_seq_indices + soff[None, :, None]).reshape(S_ROWS, G)

    mesh = plsc.VectorSubcoreMesh(core_axis_name="c", subcore_axis_name="s")
    run = pl.kernel(
        _gather_body,
        mesh=mesh,
        out_type=(
            jax.ShapeDtypeStruct((U_ROWS, G, HIDDEN), jnp.float32),
            jax.ShapeDtypeStruct((A_ROWS, G, HIDDEN), jnp.float32),
            jax.ShapeDtypeStruct((S_ROWS, G, HIDDEN), jnp.float32),
        ),
        scratch_types=[
            pltpu.VMEM((S_PW, G), jnp.int32),
            pltpu.VMEM((SK, G, HIDDEN), jnp.float32),
            pltpu.VMEM((SK, G, HIDDEN), jnp.float32),
            pltpu.VMEM((U_PW, G), jnp.int32),
            pltpu.VMEM((U_PW, G, HIDDEN), jnp.float32),
            pltpu.SemaphoreType.DMA,
            pltpu.SemaphoreType.DMA,
            pltpu.SemaphoreType.DMA,
            pltpu.SemaphoreType.DMA,
            pltpu.SemaphoreType.DMA,
            pltpu.SemaphoreType.DMA,
        ],
        compiler_params=pltpu.CompilerParams(use_tc_tiling_on_sc=False),
    )
    u_out, a_out, s_out = run(user_flat, ad_flat, uidx, aidx, sidx)
    return (
        u_out.reshape(B, N_USER, HIDDEN),
        a_out.reshape(B, N_AD, HIDDEN),
        s_out.reshape(B, len(SEQ_TABLES), HIST, HIDDEN),
    )
